# Initial kernel scaffold; baseline (speedup 1.0000x reference)
import functools

import jax
import jax.numpy as jnp
from jax import lax
from jax.experimental import pallas as pl
from jax.experimental.pallas import tpu as pltpu
from jax.experimental.pallas import tpu_sc as plsc

N_NODES = 10000
N_EDGES = 320000
D = 128

N_CORES = 2
N_SUBCORES = 16
EDGES_PER_CORE = N_EDGES // N_CORES
EDGES_PER_TILE = EDGES_PER_CORE // N_SUBCORES
CHUNK = 80
N_CHUNKS = EDGES_PER_TILE // CHUNK
ROWS_PER_TILE = N_NODES // N_SUBCORES
ZCHUNK = 125
N_ZCHUNKS = ROWS_PER_TILE // ZCHUNK
CNT_W = 16

_MESH = plsc.VectorSubcoreMesh(core_axis_name="c", subcore_axis_name="s")


def _sc_degree(dst):
    @functools.partial(
        pl.kernel,
        out_type=jax.ShapeDtypeStruct((N_CORES, N_NODES, CNT_W), jnp.float32),
        mesh=_MESH,
        scratch_types=[
            pltpu.VMEM_SHARED((N_NODES, CNT_W), jnp.float32),
            pltpu.VMEM((CHUNK,), jnp.int32),
            pltpu.VMEM((CHUNK, CNT_W), jnp.float32),
            pltpu.VMEM((ZCHUNK, CNT_W), jnp.float32),
        ],
    )
    def k(dst_hbm, out_hbm, acc, dst_v, ones_v, zeros_v):
        core = lax.axis_index("c")
        sid = lax.axis_index("s")

        @pl.loop(0, ZCHUNK)
        def _(r):
            zeros_v[r, pl.ds(0, CNT_W)] = jnp.zeros((CNT_W,), jnp.float32)

        @pl.loop(0, CHUNK)
        def _(r):
            ones_v[r, pl.ds(0, CNT_W)] = jnp.ones((CNT_W,), jnp.float32)

        row0 = sid * ROWS_PER_TILE

        @pl.loop(0, N_ZCHUNKS)
        def _(z):
            pltpu.sync_copy(zeros_v, acc.at[pl.ds(row0 + z * ZCHUNK, ZCHUNK)])

        plsc.subcore_barrier()

        ebase = core * EDGES_PER_CORE + sid * EDGES_PER_TILE

        @pl.loop(0, N_CHUNKS)
        def _(i):
            pltpu.sync_copy(dst_hbm.at[pl.ds(ebase + i * CHUNK, CHUNK)], dst_v)
            pltpu.sync_copy(ones_v, acc.at[dst_v], add=True)

        plsc.subcore_barrier()

        @pl.loop(0, N_ZCHUNKS)
        def _(z):
            r = row0 + z * ZCHUNK
            pltpu.sync_copy(acc.at[pl.ds(r, ZCHUNK)], out_hbm.at[core, pl.ds(r, ZCHUNK)])

    return k(dst)


def _sc_segsum(h, src, dst):
    @functools.partial(
        pl.kernel,
        out_type=jax.ShapeDtypeStruct((N_CORES, N_NODES, D), jnp.float32),
        mesh=_MESH,
        scratch_types=[
            pltpu.VMEM_SHARED((N_NODES, D), jnp.float32),
            pltpu.VMEM((CHUNK,), jnp.int32),
            pltpu.VMEM((CHUNK,), jnp.int32),
            pltpu.VMEM((CHUNK, D), jnp.float32),
            pltpu.VMEM((ZCHUNK, D), jnp.float32),
            pltpu.SemaphoreType.DMA,
        ],
    )
    def k(h_hbm, src_hbm, dst_hbm, out_hbm, acc, src_v, dst_v, rows_v, zeros_v, sem):
        core = lax.axis_index("c")
        sid = lax.axis_index("s")

        @pl.loop(0, ZCHUNK)
        def _(r):
            @pl.loop(0, D, step=16)
            def _(j):
                zeros_v[r, pl.ds(j, 16)] = jnp.zeros((16,), jnp.float32)

        row0 = sid * ROWS_PER_TILE

        @pl.loop(0, N_ZCHUNKS)
        def _(z):
            pltpu.sync_copy(zeros_v, acc.at[pl.ds(row0 + z * ZCHUNK, ZCHUNK)])

        plsc.subcore_barrier()

        ebase = core * EDGES_PER_CORE + sid * EDGES_PER_TILE

        @pl.loop(0, N_CHUNKS)
        def _(i):
            off = ebase + i * CHUNK
            pltpu.sync_copy(src_hbm.at[pl.ds(off, CHUNK)], src_v)
            pltpu.sync_copy(dst_hbm.at[pl.ds(off, CHUNK)], dst_v)
            pltpu.async_copy(h_hbm.at[src_v], rows_v, sem).wait()
            pltpu.sync_copy(rows_v, acc.at[dst_v], add=True)

        plsc.subcore_barrier()

        @pl.loop(0, N_ZCHUNKS)
        def _(z):
            r = row0 + z * ZCHUNK
            pltpu.sync_copy(acc.at[pl.ds(r, ZCHUNK)], out_hbm.at[core, pl.ds(r, ZCHUNK)])

    return k(h, src, dst)


_BM = 1000
_GRID = (N_NODES // _BM,)
_HIGH = lax.Precision.HIGHEST


def _dot(a, b):
    return jnp.dot(a, b, precision=_HIGH, preferred_element_type=jnp.float32)


def _tc_stage1(x, W1, cnt):
    def body(x_ref, w_ref, cnt_ref, hp_ref, dis_ref):
        c = cnt_ref[...]
        deg = 1.0 + c[0, :, :1] + c[1, :, :1]
        dis = lax.rsqrt(deg)
        hp_ref[...] = dis * _dot(x_ref[...], w_ref[...])
        dis_ref[...] = dis

    return pl.pallas_call(
        body,
        grid=_GRID,
        in_specs=[
            pl.BlockSpec((_BM, D), lambda i: (i, 0)),
            pl.BlockSpec((D, D), lambda i: (0, 0)),
            pl.BlockSpec((N_CORES, _BM, CNT_W), lambda i: (0, i, 0)),
        ],
        out_specs=[
            pl.BlockSpec((_BM, D), lambda i: (i, 0)),
            pl.BlockSpec((_BM, 1), lambda i: (i, 0)),
        ],
        out_shape=[
            jax.ShapeDtypeStruct((N_NODES, D), jnp.float32),
            jax.ShapeDtypeStruct((N_NODES, 1), jnp.float32),
        ],
    )(x, W1, cnt)


def _tc_stage2(agg, hp, dis, b, W2):
    def body(agg_ref, hp_ref, dis_ref, b_ref, w_ref, out_ref):
        a = agg_ref[0] + agg_ref[1] + hp_ref[...]
        z = jnp.maximum(dis_ref[...] * a + b_ref[...], 0.0)
        out_ref[...] = dis_ref[...] * _dot(z, w_ref[...])

    return pl.pallas_call(
        body,
        grid=_GRID,
        in_specs=[
            pl.BlockSpec((N_CORES, _BM, D), lambda i: (0, i, 0)),
            pl.BlockSpec((_BM, D), lambda i: (i, 0)),
            pl.BlockSpec((_BM, 1), lambda i: (i, 0)),
            pl.BlockSpec((1, D), lambda i: (0, 0)),
            pl.BlockSpec((D, D), lambda i: (0, 0)),
        ],
        out_specs=pl.BlockSpec((_BM, D), lambda i: (i, 0)),
        out_shape=jax.ShapeDtypeStruct((N_NODES, D), jnp.float32),
    )(agg, hp, dis, b, W2)


def _tc_stage3(agg, hp, dis, b2, Wl1, bl1, Wl2, bl2, Wl3, bl3):
    def body(agg_ref, hp_ref, dis_ref, b2_ref, w1_ref, b1_ref, w2_ref,
             bb2_ref, w3_ref, b3_ref, out_ref):
        a = agg_ref[0] + agg_ref[1] + hp_ref[...]
        z = jnp.maximum(dis_ref[...] * a + b2_ref[...], 0.0)
        t = jnp.maximum(_dot(z, w1_ref[...]) + b1_ref[...], 0.0)
        t = jnp.maximum(_dot(t, w2_ref[...]) + bb2_ref[...], 0.0)
        out_ref[...] = _dot(t, w3_ref[...]) + b3_ref[...]

    return pl.pallas_call(
        body,
        grid=_GRID,
        in_specs=[
            pl.BlockSpec((N_CORES, _BM, D), lambda i: (0, i, 0)),
            pl.BlockSpec((_BM, D), lambda i: (i, 0)),
            pl.BlockSpec((_BM, 1), lambda i: (i, 0)),
            pl.BlockSpec((1, D), lambda i: (0, 0)),
            pl.BlockSpec((D, D), lambda i: (0, 0)),
            pl.BlockSpec((1, D), lambda i: (0, 0)),
            pl.BlockSpec((D, D), lambda i: (0, 0)),
            pl.BlockSpec((1, D), lambda i: (0, 0)),
            pl.BlockSpec((D, 1), lambda i: (0, 0)),
            pl.BlockSpec((1, 1), lambda i: (0, 0)),
        ],
        out_specs=pl.BlockSpec((_BM, 1), lambda i: (i, 0)),
        out_shape=jax.ShapeDtypeStruct((N_NODES, 1), jnp.float32),
    )(agg, hp, dis, b2, Wl1, bl1, Wl2, bl2, Wl3, bl3)


def kernel(x, edge_index, W_conv1, b_conv1, W_conv2, b_conv2,
           W_lin1, b_lin1, W_lin2, b_lin2, W_lin3, b_lin3):
    src = edge_index[0].astype(jnp.int32)
    dst = edge_index[1].astype(jnp.int32)

    cnt = _sc_degree(dst)
    h1p, dis = _tc_stage1(x, W_conv1, cnt)
    agg1 = _sc_segsum(h1p, src, dst)
    h2p = _tc_stage2(agg1, h1p, dis, b_conv1.reshape(1, D), W_conv2)
    agg2 = _sc_segsum(h2p, src, dst)
    out = _tc_stage3(agg2, h2p, dis, b_conv2.reshape(1, D),
                     W_lin1, b_lin1.reshape(1, D),
                     W_lin2, b_lin2.reshape(1, D),
                     W_lin3, b_lin3.reshape(1, 1))
    return out


# speedup vs baseline: 26.5169x; 26.5169x over previous
import functools

import jax
import jax.numpy as jnp
from jax import lax
from jax.experimental import pallas as pl
from jax.experimental.pallas import tpu as pltpu
from jax.experimental.pallas import tpu_sc as plsc

N_NODES = 10000
N_EDGES = 320000
D = 128

N_CORES = 2
N_SUBCORES = 16
CHUNK = 128
E_PAD = 327680
N_CHUNKS = E_PAD // (N_CORES * N_SUBCORES * CHUNK)
IDX_ROWS = E_PAD // CHUNK
IDX_ROWS_PER_TILE = IDX_ROWS // (N_CORES * N_SUBCORES)
N_PHASES = 2
PHASE_CHUNKS = N_CHUNKS // N_PHASES
N_PAD = 10240
ROWS_PER_TILE = N_PAD // N_SUBCORES
ZCHUNK = 128
N_ZCHUNKS = ROWS_PER_TILE // ZCHUNK

_MESH = plsc.VectorSubcoreMesh(core_axis_name="c", subcore_axis_name="s")


def _sc_degree(ones_mat, dst2d):
    @functools.partial(
        pl.kernel,
        out_type=jax.ShapeDtypeStruct((N_CORES, N_PAD, D), jnp.float32),
        mesh=_MESH,
        scratch_types=[
            pltpu.VMEM_SHARED((N_PAD, D), jnp.float32),
            pltpu.VMEM((IDX_ROWS_PER_TILE, CHUNK), jnp.int32),
            pltpu.VMEM((CHUNK, D), jnp.float32),
            pltpu.SemaphoreType.DMA,
            pltpu.SemaphoreType.DMA,
        ],
    )
    def k(ones_hbm, dst_hbm, out_hbm, acc, dst_v, rows0, ssem0, ssem1):
        core = lax.axis_index("c")
        sid = lax.axis_index("s")
        ssems = (ssem0, ssem1)

        @pl.loop(0, CHUNK)
        def _(r):
            @pl.loop(0, D, step=16)
            def _(j):
                rows0[r, pl.ds(j, 16)] = jnp.zeros((16,), jnp.float32)

        row0 = sid * ROWS_PER_TILE

        @pl.loop(0, N_ZCHUNKS)
        def _(z):
            pltpu.sync_copy(rows0, acc.at[pl.ds(row0 + z * ZCHUNK, ZCHUNK)])

        irow0 = (core * N_SUBCORES + sid) * IDX_ROWS_PER_TILE
        pltpu.sync_copy(dst_hbm.at[pl.ds(irow0, IDX_ROWS_PER_TILE)], dst_v)
        pltpu.sync_copy(ones_hbm, rows0)

        plsc.subcore_barrier()

        @pl.loop(0, N_CHUNKS // 2)
        def _(g):
            for b in range(2):
                i = 2 * g + b

                @pl.when(i >= 2)
                def _():
                    pltpu.make_async_copy(rows0, acc.at[dst_v.at[i - 2]],
                                          ssems[b]).wait()

                pltpu.async_copy(rows0, acc.at[dst_v.at[i]], ssems[b], add=True)

        pltpu.make_async_copy(rows0, acc.at[dst_v.at[N_CHUNKS - 2]], ssem0).wait()
        pltpu.make_async_copy(rows0, acc.at[dst_v.at[N_CHUNKS - 1]], ssem1).wait()

        plsc.subcore_barrier()

        @pl.loop(0, N_ZCHUNKS)
        def _(z):
            r = row0 + z * ZCHUNK
            pltpu.sync_copy(acc.at[pl.ds(r, ZCHUNK)], out_hbm.at[core, pl.ds(r, ZCHUNK)])

    return k(ones_mat, dst2d)


def _sc_segsum(h, src2d, dst2d):
    @functools.partial(
        pl.kernel,
        out_type=jax.ShapeDtypeStruct((N_CORES, N_PAD, D), jnp.float32),
        mesh=_MESH,
        scratch_types=[
            pltpu.VMEM_SHARED((N_PAD, D), jnp.float32),
            pltpu.VMEM((PHASE_CHUNKS, CHUNK), jnp.int32),
            pltpu.VMEM((PHASE_CHUNKS, CHUNK), jnp.int32),
            pltpu.VMEM((CHUNK, D), jnp.float32),
            pltpu.VMEM((CHUNK, D), jnp.float32),
            pltpu.SemaphoreType.DMA,
            pltpu.SemaphoreType.DMA,
            pltpu.SemaphoreType.DMA,
            pltpu.SemaphoreType.DMA,
        ],
    )
    def k(h_hbm, src_hbm, dst_hbm, out_hbm, acc, src_v, dst_v,
          rows0, rows1, gsem0, gsem1, ssem0, ssem1):
        core = lax.axis_index("c")
        sid = lax.axis_index("s")
        rows = (rows0, rows1)
        gsems = (gsem0, gsem1)
        ssems = (ssem0, ssem1)

        @pl.loop(0, CHUNK)
        def _(r):
            @pl.loop(0, D, step=16)
            def _(j):
                rows0[r, pl.ds(j, 16)] = jnp.zeros((16,), jnp.float32)

        row0 = sid * ROWS_PER_TILE

        @pl.loop(0, N_ZCHUNKS)
        def _(z):
            pltpu.sync_copy(rows0, acc.at[pl.ds(row0 + z * ZCHUNK, ZCHUNK)])

        irow0 = (core * N_SUBCORES + sid) * IDX_ROWS_PER_TILE

        plsc.subcore_barrier()

        for p in range(N_PHASES):
            pltpu.sync_copy(
                src_hbm.at[pl.ds(irow0 + p * PHASE_CHUNKS, PHASE_CHUNKS)], src_v)
            pltpu.sync_copy(
                dst_hbm.at[pl.ds(irow0 + p * PHASE_CHUNKS, PHASE_CHUNKS)], dst_v)

            pltpu.async_copy(h_hbm.at[src_v.at[0]], rows0, gsem0)

            @pl.loop(0, PHASE_CHUNKS // 2)
            def _(g):
                for b in range(2):
                    i = 2 * g + b

                    @pl.when(i >= 1)
                    def _():
                        pltpu.make_async_copy(
                            rows[1 - b], acc.at[dst_v.at[i - 1]],
                            ssems[1 - b]).wait()

                    @pl.when(i + 1 < PHASE_CHUNKS)
                    def _():
                        pltpu.async_copy(h_hbm.at[src_v.at[i + 1]], rows[1 - b],
                                         gsems[1 - b])

                    pltpu.make_async_copy(h_hbm.at[src_v.at[i]], rows[b],
                                          gsems[b]).wait()
                    pltpu.async_copy(rows[b], acc.at[dst_v.at[i]], ssems[b],
                                     add=True)

            pltpu.make_async_copy(rows1, acc.at[dst_v.at[PHASE_CHUNKS - 1]],
                                  ssem1).wait()

        plsc.subcore_barrier()

        @pl.loop(0, N_ZCHUNKS)
        def _(z):
            r = row0 + z * ZCHUNK
            pltpu.sync_copy(acc.at[pl.ds(r, ZCHUNK)], out_hbm.at[core, pl.ds(r, ZCHUNK)])

    return k(h, src2d, dst2d)


_BM = 2000
_GRID = (N_NODES // _BM,)


def _dot(a, b):
    return jnp.dot(a, b, preferred_element_type=jnp.float32)


def _tc_matmul1(x, W1):
    def body(x_ref, w_ref, h_ref):
        h_ref[...] = _dot(x_ref[...], w_ref[...])

    return pl.pallas_call(
        body,
        grid=_GRID,
        in_specs=[
            pl.BlockSpec((_BM, D), lambda i: (i, 0)),
            pl.BlockSpec((D, D), lambda i: (0, 0)),
        ],
        out_specs=pl.BlockSpec((_BM, D), lambda i: (i, 0)),
        out_shape=jax.ShapeDtypeStruct((N_NODES, D), jnp.float32),
    )(x, W1)


def _tc_stage1(h1, cnt):
    def body(h_ref, cnt_ref, hp_ref, dis_ref):
        c = cnt_ref[...]
        deg = 1.0 + c[0, :, :1] + c[1, :, :1]
        dis = lax.rsqrt(deg)
        hp_ref[...] = dis * h_ref[...]
        dis_ref[...] = dis

    return pl.pallas_call(
        body,
        grid=_GRID,
        in_specs=[
            pl.BlockSpec((_BM, D), lambda i: (i, 0)),
            pl.BlockSpec((N_CORES, _BM, D), lambda i: (0, i, 0)),
        ],
        out_specs=[
            pl.BlockSpec((_BM, D), lambda i: (i, 0)),
            pl.BlockSpec((_BM, 1), lambda i: (i, 0)),
        ],
        out_shape=[
            jax.ShapeDtypeStruct((N_NODES, D), jnp.float32),
            jax.ShapeDtypeStruct((N_NODES, 1), jnp.float32),
        ],
    )(h1, cnt)


def _tc_stage2(agg, hp, dis, b, W2):
    def body(agg_ref, hp_ref, dis_ref, b_ref, w_ref, out_ref):
        a = agg_ref[0] + agg_ref[1] + hp_ref[...]
        z = jnp.maximum(dis_ref[...] * a + b_ref[...], 0.0)
        out_ref[...] = dis_ref[...] * _dot(z, w_ref[...])

    return pl.pallas_call(
        body,
        grid=_GRID,
        in_specs=[
            pl.BlockSpec((N_CORES, _BM, D), lambda i: (0, i, 0)),
            pl.BlockSpec((_BM, D), lambda i: (i, 0)),
            pl.BlockSpec((_BM, 1), lambda i: (i, 0)),
            pl.BlockSpec((1, D), lambda i: (0, 0)),
            pl.BlockSpec((D, D), lambda i: (0, 0)),
        ],
        out_specs=pl.BlockSpec((_BM, D), lambda i: (i, 0)),
        out_shape=jax.ShapeDtypeStruct((N_NODES, D), jnp.float32),
    )(agg, hp, dis, b, W2)


def _tc_stage3(agg, hp, dis, b2, Wl1, bl1, Wl2, bl2, Wl3, bl3):
    def body(agg_ref, hp_ref, dis_ref, b2_ref, w1_ref, b1_ref, w2_ref,
             bb2_ref, w3_ref, b3_ref, out_ref):
        a = agg_ref[0] + agg_ref[1] + hp_ref[...]
        z = jnp.maximum(dis_ref[...] * a + b2_ref[...], 0.0)
        t = jnp.maximum(_dot(z, w1_ref[...]) + b1_ref[...], 0.0)
        t = jnp.maximum(_dot(t, w2_ref[...]) + bb2_ref[...], 0.0)
        out_ref[...] = _dot(t, w3_ref[...]) + b3_ref[...]

    return pl.pallas_call(
        body,
        grid=_GRID,
        in_specs=[
            pl.BlockSpec((N_CORES, _BM, D), lambda i: (0, i, 0)),
            pl.BlockSpec((_BM, D), lambda i: (i, 0)),
            pl.BlockSpec((_BM, 1), lambda i: (i, 0)),
            pl.BlockSpec((1, D), lambda i: (0, 0)),
            pl.BlockSpec((D, D), lambda i: (0, 0)),
            pl.BlockSpec((1, D), lambda i: (0, 0)),
            pl.BlockSpec((D, D), lambda i: (0, 0)),
            pl.BlockSpec((1, D), lambda i: (0, 0)),
            pl.BlockSpec((D, 1), lambda i: (0, 0)),
            pl.BlockSpec((1, 1), lambda i: (0, 0)),
        ],
        out_specs=pl.BlockSpec((_BM, 1), lambda i: (i, 0)),
        out_shape=jax.ShapeDtypeStruct((N_NODES, 1), jnp.float32),
    )(agg, hp, dis, b2, Wl1, bl1, Wl2, bl2, Wl3, bl3)


def kernel(x, edge_index, W_conv1, b_conv1, W_conv2, b_conv2,
           W_lin1, b_lin1, W_lin2, b_lin2, W_lin3, b_lin3):
    src = edge_index[0].astype(jnp.int32)
    dst = edge_index[1].astype(jnp.int32)

    pad_i = jnp.arange(E_PAD - N_EDGES, dtype=jnp.int32)
    src_full = jnp.concatenate([src, pad_i % N_NODES]).reshape(IDX_ROWS, CHUNK)
    dst_full = jnp.concatenate([dst, N_NODES + pad_i % (N_PAD - N_NODES)]
                               ).reshape(IDX_ROWS, CHUNK)

    cnt = _sc_degree(jnp.ones((CHUNK, D), jnp.float32), dst_full)[:, :N_NODES]
    h1 = _tc_matmul1(x, W_conv1)
    h1p, dis = _tc_stage1(h1, cnt)
    agg1 = _sc_segsum(h1p, src_full, dst_full)[:, :N_NODES]
    h2p = _tc_stage2(agg1, h1p, dis, b_conv1.reshape(1, D), W_conv2)
    agg2 = _sc_segsum(h2p, src_full, dst_full)[:, :N_NODES]
    out = _tc_stage3(agg2, h2p, dis, b_conv2.reshape(1, D),
                     W_lin1, b_lin1.reshape(1, D),
                     W_lin2, b_lin2.reshape(1, D),
                     W_lin3, b_lin3.reshape(1, 1))
    return out
